# baseline (speedup 1.0000x reference)
import functools

import jax
import jax.numpy as jnp
from jax.experimental import pallas as pl


N_EXP = 8
TOP_K = 2


def _gating_body(x_ref, gw_ref, gb_ref, c_ref):
    x = x_ref[...]
    logits = jax.lax.dot_general(
        x, gw_ref[...], (((1,), (1,)), ((), ())),
        precision=jax.lax.Precision.DEFAULT,
        preferred_element_type=jnp.float32,
    ) + gb_ref[...]
    m = jnp.max(logits, axis=-1, keepdims=True)
    p = jnp.exp(logits - m)
    s = p / jnp.sum(p, axis=-1, keepdims=True)
    ii = jax.lax.broadcasted_iota(jnp.int32, s.shape, 1)
    m1 = jnp.max(s, axis=-1, keepdims=True)
    i1 = jnp.min(jnp.where(s == m1, ii, N_EXP), axis=-1, keepdims=True)
    s2 = jnp.where(ii == i1, -jnp.inf, s)
    m2 = jnp.max(s2, axis=-1, keepdims=True)
    i2 = jnp.min(jnp.where(s2 == m2, ii, N_EXP), axis=-1, keepdims=True)
    sel = (ii == i1) | (ii == i2)
    c_ref[...] = jnp.where(sel, s, 0.0) * (1.0 / TOP_K)


def _moe_body(c_ref, x_ref, w_ref, b_ref, out_ref):
    e = pl.program_id(1)
    c = c_ref[...]
    ee = jax.lax.broadcasted_iota(jnp.int32, c.shape, 1)
    c_col = jnp.sum(jnp.where(ee == e, c, 0.0), axis=1, keepdims=True)
    xb = x_ref[...].astype(jnp.bfloat16)
    wb = w_ref[0].astype(jnp.bfloat16)
    t = jnp.dot(xb, wb, preferred_element_type=jnp.float32)
    t = (t + b_ref[0]) * c_col

    @pl.when(e == 0)
    def _init():
        out_ref[...] = t

    @pl.when(e != 0)
    def _acc():
        out_ref[...] += t


def kernel(sequences, expert_weights, expert_biases, gating_w, gating_b):
    n, p, d = sequences.shape
    tokens = n * p
    d_inner = expert_biases.shape[-1]
    x = sequences.reshape(tokens, d)

    c = pl.pallas_call(
        _gating_body,
        out_shape=jax.ShapeDtypeStruct((tokens, N_EXP), jnp.float32),
    )(x, gating_w, gating_b.reshape(1, N_EXP))

    tn = 512
    n_tiles = d_inner // tn
    out = pl.pallas_call(
        _moe_body,
        grid=(n_tiles, N_EXP),
        in_specs=[
            pl.BlockSpec((tokens, N_EXP), lambda ni, e: (0, 0)),
            pl.BlockSpec((tokens, d), lambda ni, e: (0, 0)),
            pl.BlockSpec((1, d, tn), lambda ni, e: (e, 0, ni)),
            pl.BlockSpec((1, 1, tn), lambda ni, e: (e, 0, ni)),
        ],
        out_specs=pl.BlockSpec((tokens, tn), lambda ni, e: (0, ni)),
        out_shape=jax.ShapeDtypeStruct((tokens, d_inner), jnp.float32),
    )(c, x, expert_weights, expert_biases.reshape(N_EXP, 1, d_inner))

    return out.reshape(n, p, d_inner)


# speedup vs baseline: 1.0689x; 1.0689x over previous
import jax
import jax.numpy as jnp
from jax.experimental import pallas as pl


N_EXP = 8
TOP_K = 2


def _gating_body(x_ref, gw_ref, gb_ref, c_ref, xc_ref):
    x = x_ref[...]
    logits = jax.lax.dot_general(
        x, gw_ref[...], (((1,), (1,)), ((), ())),
        precision=jax.lax.Precision.DEFAULT,
        preferred_element_type=jnp.float32,
    ) + gb_ref[...]
    m = jnp.max(logits, axis=-1, keepdims=True)
    p = jnp.exp(logits - m)
    s = p / jnp.sum(p, axis=-1, keepdims=True)
    ii = jax.lax.broadcasted_iota(jnp.int32, s.shape, 1)
    m1 = jnp.max(s, axis=-1, keepdims=True)
    i1 = jnp.min(jnp.where(s == m1, ii, N_EXP), axis=-1, keepdims=True)
    s2 = jnp.where(ii == i1, -jnp.inf, s)
    m2 = jnp.max(s2, axis=-1, keepdims=True)
    i2 = jnp.min(jnp.where(s2 == m2, ii, N_EXP), axis=-1, keepdims=True)
    sel = (ii == i1) | (ii == i2)
    c = jnp.where(sel, s, 0.0) * (1.0 / TOP_K)
    c_ref[...] = c
    d = x_ref.shape[1]
    for e in range(N_EXP):
        xc_ref[:, e * d:(e + 1) * d] = (x * c[:, e:e + 1]).astype(jnp.bfloat16)


def _moe_body(c_ref, xc_ref, w_ref, b_ref, out_ref):
    t = jnp.dot(xc_ref[...], w_ref[...].astype(jnp.bfloat16),
                preferred_element_type=jnp.float32)
    t += jnp.dot(c_ref[...], b_ref[...], preferred_element_type=jnp.float32)
    out_ref[...] = t


def kernel(sequences, expert_weights, expert_biases, gating_w, gating_b):
    n, p, d = sequences.shape
    tokens = n * p
    d_inner = expert_biases.shape[-1]
    x = sequences.reshape(tokens, d)
    k_all = N_EXP * d

    c, xc = pl.pallas_call(
        _gating_body,
        out_shape=[
            jax.ShapeDtypeStruct((tokens, N_EXP), jnp.float32),
            jax.ShapeDtypeStruct((tokens, k_all), jnp.bfloat16),
        ],
    )(x, gating_w, gating_b.reshape(1, N_EXP))

    tn = 256
    tm = tokens // 2
    n_tiles = d_inner // tn
    out = pl.pallas_call(
        _moe_body,
        grid=(2, n_tiles),
        in_specs=[
            pl.BlockSpec((tm, N_EXP), lambda mi, ni: (mi, 0)),
            pl.BlockSpec((tm, k_all), lambda mi, ni: (mi, 0)),
            pl.BlockSpec((k_all, tn), lambda mi, ni: (0, ni)),
            pl.BlockSpec((N_EXP, tn), lambda mi, ni: (0, ni)),
        ],
        out_specs=pl.BlockSpec((tm, tn), lambda mi, ni: (mi, ni)),
        out_shape=jax.ShapeDtypeStruct((tokens, d_inner), jnp.float32),
    )(c, xc, expert_weights.reshape(k_all, d_inner), expert_biases)

    return out.reshape(n, p, d_inner)


# speedup vs baseline: 1.1834x; 1.1071x over previous
import jax
import jax.numpy as jnp
from jax.experimental import pallas as pl
from jax.experimental.pallas import tpu as pltpu


N_EXP = 8
TOP_K = 2


def _moe_body(x_ref, gw_ref, gb_ref, w_ref, b_ref, out_ref, xc_ref, c_ref):
    d = x_ref.shape[1]

    @pl.when(pl.program_id(1) == 0)
    def _gate():
        x = x_ref[...]
        logits = jax.lax.dot_general(
            x, gw_ref[...], (((1,), (1,)), ((), ())),
            precision=jax.lax.Precision.DEFAULT,
            preferred_element_type=jnp.float32,
        ) + gb_ref[...]
        m = jnp.max(logits, axis=-1, keepdims=True)
        p = jnp.exp(logits - m)
        s = p / jnp.sum(p, axis=-1, keepdims=True)
        ii = jax.lax.broadcasted_iota(jnp.int32, s.shape, 1)
        m1 = jnp.max(s, axis=-1, keepdims=True)
        i1 = jnp.min(jnp.where(s == m1, ii, N_EXP), axis=-1, keepdims=True)
        s2 = jnp.where(ii == i1, -jnp.inf, s)
        m2 = jnp.max(s2, axis=-1, keepdims=True)
        i2 = jnp.min(jnp.where(s2 == m2, ii, N_EXP), axis=-1, keepdims=True)
        sel = (ii == i1) | (ii == i2)
        c = jnp.where(sel, s, 0.0) * (1.0 / TOP_K)
        c_ref[...] = c
        for e in range(N_EXP):
            xc_ref[:, e * d:(e + 1) * d] = (x * c[:, e:e + 1]).astype(jnp.bfloat16)

    t = jnp.dot(xc_ref[...], w_ref[...].astype(jnp.bfloat16),
                preferred_element_type=jnp.float32)
    t += jnp.dot(c_ref[...], b_ref[...], preferred_element_type=jnp.float32)
    out_ref[...] = t


def kernel(sequences, expert_weights, expert_biases, gating_w, gating_b):
    n, p, d = sequences.shape
    tokens = n * p
    d_inner = expert_biases.shape[-1]
    x = sequences.reshape(tokens, d)
    k_all = N_EXP * d

    tn = 256
    tm = tokens // 2
    n_tiles = d_inner // tn
    out = pl.pallas_call(
        _moe_body,
        grid=(2, n_tiles),
        in_specs=[
            pl.BlockSpec((tm, d), lambda mi, ni: (mi, 0)),
            pl.BlockSpec((N_EXP, d), lambda mi, ni: (0, 0)),
            pl.BlockSpec((1, N_EXP), lambda mi, ni: (0, 0)),
            pl.BlockSpec((k_all, tn), lambda mi, ni: (0, ni)),
            pl.BlockSpec((N_EXP, tn), lambda mi, ni: (0, ni)),
        ],
        out_specs=pl.BlockSpec((tm, tn), lambda mi, ni: (mi, ni)),
        out_shape=jax.ShapeDtypeStruct((tokens, d_inner), jnp.float32),
        scratch_shapes=[
            pltpu.VMEM((tm, k_all), jnp.bfloat16),
            pltpu.VMEM((tm, N_EXP), jnp.float32),
        ],
    )(x, gating_w, gating_b.reshape(1, N_EXP), expert_weights.reshape(k_all, d_inner), expert_biases)

    return out.reshape(n, p, d_inner)
